# baseline (speedup 1.0000x reference)
import functools

import numpy as np
import jax
import jax.numpy as jnp
from jax import lax
from jax.experimental import pallas as pl
from jax.experimental.pallas import tpu as pltpu
from jax.experimental.pallas import tpu_sc as plsc

N = 10000
E = 320000
CH = 128
HALF = 64
NC, NS, L = 2, 16, 16
B = 128
NBATCH = 162
T = NBATCH * B
E_PAD = NS * T
NBT = E_PAD // B
ROWS_PER_TILE = (2 * N) // NS
DEG_CHUNK = 2000

def _pack_x(x):
    R = 2000

    def body(x_ref, o_ref):
        xr = x_ref[...].reshape(R // 2, 2, CH)
        parts = [xr[:, 0, :], xr[:, 1, :]]
        groups = []
        for p in parts:
            u = lax.bitcast_convert_type(p, jnp.uint32)
            r = (u + 0x7FFF + ((u >> 16) & 1)) >> 16
            for c in range(2):
                lo = jnp.concatenate(
                    [r[:, 64 * c:64 * c + 16], r[:, 64 * c + 32:64 * c + 48]],
                    axis=1)
                hi = jnp.concatenate(
                    [r[:, 64 * c + 16:64 * c + 32], r[:, 64 * c + 48:64 * c + 64]],
                    axis=1)
                groups.append(lo | (hi << 16))
        o_ref[...] = lax.bitcast_convert_type(
            jnp.concatenate(groups, axis=1), jnp.int32)

    return pl.pallas_call(
        body,
        grid=(N // R,),
        in_specs=[pl.BlockSpec((R, CH), lambda i: (i, 0))],
        out_specs=pl.BlockSpec((R // 2, CH), lambda i: (i, 0)),
        out_shape=jax.ShapeDtypeStruct((N // 2, CH), jnp.int32),
    )(x)


def _sc_aggregate(x2, edata):
    mesh = plsc.VectorSubcoreMesh(
        core_axis_name="c", subcore_axis_name="s", num_cores=NC, num_subcores=NS
    )

    @functools.partial(
        pl.kernel,
        out_type=[
            jax.ShapeDtypeStruct((NC, NS, ROWS_PER_TILE, HALF), jnp.float32),
            jax.ShapeDtypeStruct((NC, 10, DEG_CHUNK), jnp.float32),
        ],
        mesh=mesh,
        compiler_params=pltpu.CompilerParams(
            needs_layout_passes=False, use_tc_tiling_on_sc=False),
        scratch_types=[
            pltpu.VMEM_SHARED((2 * N, HALF), jnp.float32),
            pltpu.VMEM_SHARED((2 * N,), jnp.float32),
            pltpu.VMEM((3, 3, B), jnp.int32),
            pltpu.VMEM((3, B), jnp.int32),
            pltpu.VMEM((3, B), jnp.int32),
            pltpu.VMEM((3, B), jnp.float32),
            pltpu.VMEM((3, B, HALF // 2), jnp.int32),
            pltpu.VMEM((3, B, HALF), jnp.float32),
            pltpu.VMEM((DEG_CHUNK,), jnp.float32),
            pltpu.SemaphoreType.DMA,
            pltpu.SemaphoreType.DMA,
            pltpu.SemaphoreType.DMA,
            pltpu.SemaphoreType.DMA,
            pltpu.SemaphoreType.DMA,
            pltpu.SemaphoreType.DMA,
            pltpu.SemaphoreType.DMA,
            pltpu.SemaphoreType.DMA,
            pltpu.SemaphoreType.DMA,
            pltpu.SemaphoreType.DMA,
            pltpu.SemaphoreType.DMA,
            pltpu.SemaphoreType.DMA,
        ],
    )
    def sc_kernel(x2_hbm, edata_hbm, acc_out, deg_out,
                  acc_sh, deg_sh, ebuf_v, ridx_v, sidx_v, w_v, rbf_v, rf_v,
                  zd_v, t0s, t1s, t2s, g0, g1, g2, s0, s1, s2, d0, d1, d2):
        cid = lax.axis_index("c")
        sid = lax.axis_index("s")
        tsem = (t0s, t1s, t2s)
        gsem = (g0, g1, g2)
        ssem = (s0, s1, s2)
        dsem = (d0, d1, d2)

        zero16 = jnp.zeros((L,), jnp.float32)
        for r in range(B):
            for j in range(HALF // L):
                rf_v[0, r, pl.ds(j * L, L)] = zero16
        for j in range(DEG_CHUNK // L):
            zd_v[pl.ds(j * L, L)] = zero16
        r0 = sid * ROWS_PER_TILE
        for k in range(9):
            pltpu.sync_copy(rf_v.at[0], acc_sh.at[pl.ds(r0 + k * B, B)])
        rem = ROWS_PER_TILE - 9 * B
        pltpu.sync_copy(rf_v.at[0, pl.ds(0, rem)],
                        acc_sh.at[pl.ds(r0 + 9 * B, rem)])

        @pl.when(sid < 10)
        def _zero_deg():
            pltpu.sync_copy(zd_v, deg_sh.at[pl.ds(sid * DEG_CHUNK, DEG_CHUNK)])

        plsc.subcore_barrier()

        tb = sid * NBATCH

        def start_stage(bank, t):
            pltpu.async_copy(edata_hbm.at[t], ebuf_v.at[bank], tsem[bank])

        def wait_stage(bank, t):
            pltpu.make_async_copy(edata_hbm.at[t], ebuf_v.at[bank],
                                  tsem[bank]).wait()

        def compute_idx(bank):
            for g in range(B // L):
                gl = pl.ds(g * L, L)
                s16 = ebuf_v[bank, 0, gl]
                d16 = ebuf_v[bank, 1, gl]
                a16 = plsc.bitcast(ebuf_v[bank, 2, gl], jnp.float32)
                ridx_v[bank, gl] = 2 * s16 + cid
                sidx_v[bank, gl] = d16 + jnp.where(a16 < 0.0, N, 0)
                w_v[bank, gl] = jnp.abs(a16)

        def start_gather(bank):
            pltpu.async_copy(x2_hbm.at[ridx_v.at[bank]], rbf_v.at[bank],
                             gsem[bank])

        def wait_gather(bank):
            pltpu.make_async_copy(x2_hbm.at[ridx_v.at[bank]],
                                  rbf_v.at[bank], gsem[bank]).wait()

        def scale(bank):
            mask_hi = jnp.full((L,), -65536, jnp.int32)

            def group(g, carry):
                gl = pl.ds(g * L, L)
                a16 = plsc.bitcast(ebuf_v[bank, 2, gl], jnp.float32)
                w16 = jnp.abs(a16)
                for blk in range(L // 4):
                    es = [g * L + blk * 4 + u for u in range(4)]
                    wvs = [w16.at[jnp.full((L,), blk * 4 + u, jnp.int32)]
                           .get(mode="promise_in_bounds") for u in range(4)]
                    packed = [
                        [rbf_v[bank, e, pl.ds(q * L, L)] for q in range(2)]
                        for e in es
                    ]
                    for u in range(4):
                        for q in range(2):
                            p = packed[u][q]
                            lo = plsc.bitcast(p << 16, jnp.float32)
                            hi = plsc.bitcast(p & mask_hi, jnp.float32)
                            base = q * 2 * L
                            rf_v[bank, es[u], pl.ds(base, L)] = lo * wvs[u]
                            rf_v[bank, es[u], pl.ds(base + L, L)] = hi * wvs[u]
                return carry
            lax.fori_loop(0, B // L, group, 0)

        def start_scatter(bank):
            pltpu.async_copy(rf_v.at[bank], acc_sh.at[sidx_v.at[bank]],
                             ssem[bank], add=True)
            pltpu.async_copy(w_v.at[bank], deg_sh.at[sidx_v.at[bank]],
                             dsem[bank], add=True)

        def wait_scatter(bank):
            pltpu.make_async_copy(rf_v.at[bank],
                                  acc_sh.at[sidx_v.at[bank]],
                                  ssem[bank]).wait()
            pltpu.make_async_copy(w_v.at[bank],
                                  deg_sh.at[sidx_v.at[bank]],
                                  dsem[bank]).wait()

        def section(t, cur, nxt, pre, first):
            start_stage(pre, t + 2)
            wait_stage(nxt, t + 1)
            if not first:
                wait_scatter(nxt)
            compute_idx(nxt)
            start_gather(nxt)
            wait_gather(cur)
            scale(cur)
            start_scatter(cur)

        pltpu.sync_copy(edata_hbm.at[tb], ebuf_v.at[0])
        compute_idx(0)
        start_gather(0)
        start_stage(1, tb + 1)
        section(tb + 0, 0, 1, 2, True)
        section(tb + 1, 1, 2, 0, True)
        section(tb + 2, 2, 0, 1, False)

        def body(h, carry):
            t0 = tb + 3 * h
            section(t0 + 0, 0, 1, 2, False)
            section(t0 + 1, 1, 2, 0, False)
            section(t0 + 2, 2, 0, 1, False)
            return carry

        lax.fori_loop(1, NBATCH // 3, body, 0)

        wait_gather(0)
        wait_scatter(1)
        wait_scatter(2)
        wait_stage(1, tb + NBATCH + 1)
        plsc.subcore_barrier()

        pltpu.sync_copy(acc_sh.at[pl.ds(r0, ROWS_PER_TILE)],
                        acc_out.at[cid, sid])

        @pl.when(sid < 10)
        def _write_deg():
            d0_ = sid * DEG_CHUNK
            pltpu.sync_copy(deg_sh.at[pl.ds(d0_, DEG_CHUNK)],
                            deg_out.at[cid, sid])

    return sc_kernel(x2, edata)


def _tc_dense(acc, deg, xp, W_pos_l, W_pos_r, b_pos, W_neg_l, W_neg_r, b_neg):
    RP = 1000
    grid = (N // 2 // RP,)

    def body(a_ref, deg_ref, x_ref, wpl, wpr, bp, wnl, wnr, bn, o_ref):
        a = a_ref[...]
        dg = deg_ref[0]
        xb = x_ref[...]
        dims = (((1,), (1,)), ((), ()))

        def branch(b, wl, wr, bias):
            outs = []
            for par in range(2):
                agg = jnp.concatenate(
                    [a[0, b][:, 64 * par:64 * par + 64],
                     a[1, b][:, 64 * par:64 * par + 64]], axis=-1)
                d = jnp.where(dg[b, par] > 0.0, dg[b, par], 1.0)[:, None]
                xe = xb[:, 128 * par:128 * par + 128]
                o = (lax.dot_general(agg / d, wl[...], dims,
                                     preferred_element_type=jnp.float32)
                     + lax.dot_general(xe, wr[...], dims,
                                       preferred_element_type=jnp.float32)
                     + bias[...])
                outs.append(o)
            return outs

        op_e, op_o = branch(0, wpl, wpr, bp)
        on_e, on_o = branch(1, wnl, wnr, bn)
        o_ref[...] = jnp.maximum(
            jnp.concatenate([op_e, on_e, op_o, on_o], axis=-1), 0.0)

    return pl.pallas_call(
        body,
        grid=grid,
        in_specs=[
            pl.BlockSpec((2, 2, RP, CH), lambda i: (0, 0, i, 0)),
            pl.BlockSpec((1, 2, 2, RP), lambda i: (i, 0, 0, 0)),
            pl.BlockSpec((RP, 2 * CH), lambda i: (i, 0)),
            pl.BlockSpec((CH, CH), lambda i: (0, 0)),
            pl.BlockSpec((CH, CH), lambda i: (0, 0)),
            pl.BlockSpec((1, CH), lambda i: (0, 0)),
            pl.BlockSpec((CH, CH), lambda i: (0, 0)),
            pl.BlockSpec((CH, CH), lambda i: (0, 0)),
            pl.BlockSpec((1, CH), lambda i: (0, 0)),
        ],
        out_specs=pl.BlockSpec((RP, 4 * CH), lambda i: (i, 0)),
        out_shape=jax.ShapeDtypeStruct((N // 2, 4 * CH), jnp.float32),
    )(acc, deg, xp, W_pos_l, W_pos_r, b_pos.reshape(1, CH),
      W_neg_l, W_neg_r, b_neg.reshape(1, CH))


def kernel(x, edge_index, edge_attr, W_pos_l, W_pos_r, b_pos,
           W_neg_l, W_neg_r, b_neg):
    src = edge_index[0].astype(jnp.int32)
    dst = edge_index[1].astype(jnp.int32)
    attr_bits = lax.bitcast_convert_type(edge_attr, jnp.int32)
    pad = E_PAD - E
    packed = jnp.stack([
        jnp.pad(src, (0, pad)),
        jnp.pad(dst, (0, pad)),
        jnp.pad(attr_bits, (0, pad)),
    ])
    edata = packed.reshape(3, NBT, B).transpose(1, 0, 2)
    edata = jnp.pad(edata, ((0, 2), (0, 0), (0, 0)))
    x2 = _pack_x(x).reshape(2 * N, HALF // 2)

    acc, deg = _sc_aggregate(x2, edata)
    acc = acc.reshape(NC, 2, N // 2, 2 * HALF)
    deg = deg[0].reshape(2, N // 2, 2).transpose(0, 2, 1)
    deg = deg.reshape(2, 2, 5, 1000).transpose(2, 0, 1, 3)
    xp = x.reshape(N // 2, 2 * CH)
    out = _tc_dense(acc, deg, xp, W_pos_l, W_pos_r, b_pos,
                    W_neg_l, W_neg_r, b_neg)
    return out.reshape(N, 2 * CH)
